# baseline (speedup 1.0000x reference)
import functools

import jax
import jax.numpy as jnp
from jax.experimental import pallas as pl
from jax.experimental.pallas import tpu as pltpu

_CODEBOOK = 1024
_DIM = 256
_BETA = 0.25
_ROWS = 16384
_BLK = 1024
_NBLK = _ROWS // _BLK


def _vq_block_kernel(z_ref, w_ref, z2_ref, w2_ref, zq_ref, idx_ref, loss_ref):
    i = pl.program_id(0)
    zb = z_ref[...]
    w = w_ref[...]
    z2 = z2_ref[...]
    w2 = w2_ref[...]

    mm = jax.lax.dot_general(
        zb, w, (((1,), (1,)), ((), ())),
        preferred_element_type=jnp.float32)

    d = z2 + w2 - 2.0 * mm

    dmin = jnp.min(d, axis=1, keepdims=True)
    cols = jax.lax.broadcasted_iota(jnp.int32, (_BLK, _CODEBOOK), 1)
    idx = jnp.min(jnp.where(d == dmin, cols, _CODEBOOK), axis=1)
    idx = idx.astype(jnp.int32)
    idx_ref[0, 0, :] = idx

    onehot = (cols == idx[:, None]).astype(jnp.float32)
    zq = jax.lax.dot_general(
        onehot, w, (((1,), (0,)), ((), ())),
        precision=jax.lax.Precision.HIGHEST,
        preferred_element_type=jnp.float32)

    diff = zq - zb
    zq_ref[...] = zb + diff

    @pl.when(i == 0)
    def _():
        loss_ref[...] = jnp.zeros((1, 1), jnp.float32)
    loss_ref[...] += jnp.sum(diff * diff).reshape(1, 1)


@functools.partial(jax.jit)
def kernel(z, W):
    B, C, T, H, Wd = z.shape
    zt = jnp.transpose(z, (0, 2, 3, 4, 1))
    z_flat = zt.reshape(_ROWS, _DIM)

    z2x = (z_flat ** 2).sum(axis=1, keepdims=True)
    w2x = (W ** 2).sum(axis=1)

    zq_flat, idx3, loss_sum = pl.pallas_call(
        _vq_block_kernel,
        grid=(_NBLK,),
        in_specs=[
            pl.BlockSpec((_BLK, _DIM), lambda i: (i, 0)),
            pl.BlockSpec((_CODEBOOK, _DIM), lambda i: (0, 0)),
            pl.BlockSpec((_BLK, 1), lambda i: (i, 0)),
            pl.BlockSpec((1, _CODEBOOK), lambda i: (0, 0)),
        ],
        out_specs=[
            pl.BlockSpec((_BLK, _DIM), lambda i: (i, 0)),
            pl.BlockSpec((1, 1, _BLK), lambda i: (i, 0, 0)),
            pl.BlockSpec((1, 1), lambda i: (0, 0)),
        ],
        out_shape=[
            jax.ShapeDtypeStruct((_ROWS, _DIM), jnp.float32),
            jax.ShapeDtypeStruct((_NBLK, 1, _BLK), jnp.int32),
            jax.ShapeDtypeStruct((1, 1), jnp.float32),
        ],
        compiler_params=pltpu.CompilerParams(
            dimension_semantics=("arbitrary",)),
    )(z_flat, W, z2x, w2x.reshape(1, _CODEBOOK))

    d_argmin = idx3.reshape(_ROWS)
    mean_sq = loss_sum[0, 0] / (_ROWS * _DIM)
    loss = mean_sq + _BETA * mean_sq
    z_q = jnp.transpose(zq_flat.reshape(B, T, H, Wd, C), (0, 4, 1, 2, 3))
    return (z_q, loss, d_argmin)
